# baseline (speedup 1.0000x reference)
import functools

import jax
import jax.numpy as jnp
import numpy as np
from jax import lax
from jax.experimental import pallas as pl
from jax.experimental.pallas import tpu as pltpu

_ATOMS = 51
_V_MAX = 10.0
_V_MIN = -10.0
_SKEW = 0.0

_BM = 2048
_NBUF = 2


def _proj_matrix():
    atoms = _ATOMS
    delta = np.float32((_V_MAX - _V_MIN) / (atoms - 1))
    supports = np.linspace(_V_MIN, _V_MAX, atoms).astype(np.float32)
    tz = np.clip(np.float32(_SKEW) + supports, _V_MIN, _V_MAX).astype(np.float32)
    b = ((tz - np.float32(_V_MIN)) / delta).astype(np.float32)
    l = np.floor(b)
    u = np.ceil(b)
    l = np.where((u > 0) & (l == u), l - 1.0, l).astype(np.float32)
    u = np.where((l < atoms - 1) & (l == u), u + 1.0, u).astype(np.float32)
    w = np.zeros((atoms, atoms), dtype=np.float32)
    for j in range(atoms):
        w[j, int(l[j])] += np.float32(u[j] - b[j])
        w[j, int(u[j])] += np.float32(b[j] - l[j])
    return w


def _body(batch, a_hbm, f_hbm, w_ref, out_ref, a_buf, f_buf, a_sem, f_sem):
    nchunks = batch // _BM

    def start(i, slot):
        rows = pl.ds(i * _BM, _BM)
        pltpu.make_async_copy(a_hbm.at[rows], a_buf.at[slot], a_sem.at[slot]).start()
        pltpu.make_async_copy(f_hbm.at[rows], f_buf.at[slot], f_sem.at[slot]).start()

    def wait(i, slot):
        rows = pl.ds(i * _BM, _BM)
        pltpu.make_async_copy(a_hbm.at[rows], a_buf.at[slot], a_sem.at[slot]).wait()
        pltpu.make_async_copy(f_hbm.at[rows], f_buf.at[slot], f_sem.at[slot]).wait()

    for s in range(_NBUF):
        start(s, s)

    def step(i, acc):
        slot = lax.rem(i, _NBUF)
        wait(i, slot)
        logf = jnp.log(f_buf[slot] + 1e-16)
        proj = lax.dot_general(
            a_buf[slot], w_ref[...],
            dimension_numbers=(((1,), (0,)), ((), ())),
            preferred_element_type=jnp.float32,
        )
        acc = acc + jnp.sum(proj * logf, keepdims=True)

        @pl.when(i + _NBUF < nchunks)
        def _():
            start(i + _NBUF, slot)

        return acc

    acc = lax.fori_loop(0, nchunks, step, jnp.zeros((1, 1), jnp.float32))
    out_ref[...] = acc


def kernel(anchor, feature):
    batch, atoms = anchor.shape
    w = jnp.asarray(_proj_matrix())
    total = pl.pallas_call(
        functools.partial(_body, batch),
        in_specs=[
            pl.BlockSpec(memory_space=pl.ANY),
            pl.BlockSpec(memory_space=pl.ANY),
            pl.BlockSpec(memory_space=pltpu.VMEM),
        ],
        out_specs=pl.BlockSpec(memory_space=pltpu.VMEM),
        out_shape=jax.ShapeDtypeStruct((1, 1), jnp.float32),
        scratch_shapes=[
            pltpu.VMEM((_NBUF, _BM, atoms), jnp.float32),
            pltpu.VMEM((_NBUF, _BM, atoms), jnp.float32),
            pltpu.SemaphoreType.DMA((_NBUF,)),
            pltpu.SemaphoreType.DMA((_NBUF,)),
        ],
    )(anchor, feature, w)
    return (-total[0, 0] / batch).astype(jnp.float32)


# speedup vs baseline: 1.0984x; 1.0984x over previous
import functools

import jax
import jax.numpy as jnp
import numpy as np
from jax import lax
from jax.experimental import pallas as pl
from jax.experimental.pallas import tpu as pltpu

_ATOMS = 51
_V_MAX = 10.0
_V_MIN = -10.0
_SKEW = 0.0

_BM = 2048
_NBUF = 4


def _proj_matrix():
    atoms = _ATOMS
    delta = np.float32((_V_MAX - _V_MIN) / (atoms - 1))
    supports = np.linspace(_V_MIN, _V_MAX, atoms).astype(np.float32)
    tz = np.clip(np.float32(_SKEW) + supports, _V_MIN, _V_MAX).astype(np.float32)
    b = ((tz - np.float32(_V_MIN)) / delta).astype(np.float32)
    l = np.floor(b)
    u = np.ceil(b)
    l = np.where((u > 0) & (l == u), l - 1.0, l).astype(np.float32)
    u = np.where((l < atoms - 1) & (l == u), u + 1.0, u).astype(np.float32)
    w = np.zeros((atoms, atoms), dtype=np.float32)
    for j in range(atoms):
        w[j, int(l[j])] += np.float32(u[j] - b[j])
        w[j, int(u[j])] += np.float32(b[j] - l[j])
    return w


def _body(batch, a_hbm, f_hbm, w_ref, out_ref, a_buf, f_buf, a_sem, f_sem):
    nchunks = batch // _BM

    def start(i, slot):
        rows = pl.ds(i * _BM, _BM)
        pltpu.make_async_copy(a_hbm.at[rows], a_buf.at[slot], a_sem.at[slot]).start()
        pltpu.make_async_copy(f_hbm.at[rows], f_buf.at[slot], f_sem.at[slot]).start()

    def wait(i, slot):
        rows = pl.ds(i * _BM, _BM)
        pltpu.make_async_copy(a_hbm.at[rows], a_buf.at[slot], a_sem.at[slot]).wait()
        pltpu.make_async_copy(f_hbm.at[rows], f_buf.at[slot], f_sem.at[slot]).wait()

    for s in range(_NBUF):
        start(s, s)

    def step(i, acc):
        slot = lax.rem(i, _NBUF)
        wait(i, slot)
        logf = jnp.log(f_buf[slot] + 1e-16)
        proj = lax.dot_general(
            a_buf[slot], w_ref[...],
            dimension_numbers=(((1,), (0,)), ((), ())),
            preferred_element_type=jnp.float32,
        )
        acc = acc + jnp.sum(proj * logf, keepdims=True)

        @pl.when(i + _NBUF < nchunks)
        def _():
            start(i + _NBUF, slot)

        return acc

    acc = lax.fori_loop(0, nchunks, step, jnp.zeros((1, 1), jnp.float32))
    out_ref[...] = acc


def kernel(anchor, feature):
    batch, atoms = anchor.shape
    w = jnp.asarray(_proj_matrix())
    total = pl.pallas_call(
        functools.partial(_body, batch),
        in_specs=[
            pl.BlockSpec(memory_space=pl.ANY),
            pl.BlockSpec(memory_space=pl.ANY),
            pl.BlockSpec(memory_space=pltpu.VMEM),
        ],
        out_specs=pl.BlockSpec(memory_space=pltpu.VMEM),
        out_shape=jax.ShapeDtypeStruct((1, 1), jnp.float32),
        scratch_shapes=[
            pltpu.VMEM((_NBUF, _BM, atoms), jnp.float32),
            pltpu.VMEM((_NBUF, _BM, atoms), jnp.float32),
            pltpu.SemaphoreType.DMA((_NBUF,)),
            pltpu.SemaphoreType.DMA((_NBUF,)),
        ],
    )(anchor, feature, w)
    return (-total[0, 0] / batch).astype(jnp.float32)
